# Initial kernel scaffold; baseline (speedup 1.0000x reference)
import jax
import jax.numpy as jnp
from jax.experimental import pallas as pl


def kernel(x, mask, weight, bias):
    raise NotImplementedError("write your pallas kernel here")


# speedup vs baseline: 1.9198x; 1.9198x over previous
import functools

import jax
import jax.numpy as jnp
from jax.experimental import pallas as pl
from jax.experimental.pallas import tpu as pltpu

B, CIN, COUT, H, W, K = 2, 96, 96, 384, 384, 3
HW = H * W
S = 8192
NT = HW // S
HALO = 512


def _conv_body(w_ref, b_ref, xl_ref, xc_ref, xr_ref, m_ref, o_ref):
    i = pl.program_id(1)
    zeros_halo = jnp.zeros((CIN, HALO), jnp.float32)
    xl = jnp.where(i == 0, zeros_halo, xl_ref[0])
    xr = jnp.where(i == NT - 1, zeros_halo, xr_ref[0])
    xcat = jnp.concatenate([xl, xc_ref[0], xr], axis=1)

    j = jax.lax.broadcasted_iota(jnp.int32, (1, S + 2 * HALO), 1)
    col = (i * S + j + (2 * W - HALO)) % W
    x_m1 = jnp.where(col == W - 1, 0.0, xcat)
    x_p1 = jnp.where(col == 0, 0.0, xcat)

    acc = jnp.zeros((COUT, S), jnp.float32)
    for ky in range(K):
        for kx in range(K):
            src = (x_m1, xcat, x_p1)[kx]
            d = (ky - 1) * W + (kx - 1)
            sl = jax.lax.slice(src, (0, HALO + d), (CIN, HALO + d + S))
            acc = acc + jnp.dot(w_ref[ky * K + kx], sl,
                                preferred_element_type=jnp.float32)

    m = m_ref[0].astype(jnp.float32)
    o_ref[0] = (acc + b_ref[...]) * m


@jax.jit
def kernel(x, mask, weight, bias):
    xf = x.reshape(B, CIN, HW)
    mf = mask.reshape(B * NT, 1, S)
    wt = weight.transpose(2, 3, 0, 1).reshape(K * K, COUT, CIN)
    b2 = bias.reshape(COUT, 1)

    grid = (B, NT)
    out = pl.pallas_call(
        _conv_body,
        grid=grid,
        in_specs=[
            pl.BlockSpec((K * K, COUT, CIN), lambda b, i: (0, 0, 0)),
            pl.BlockSpec((COUT, 1), lambda b, i: (0, 0)),
            pl.BlockSpec((1, CIN, HALO),
                         lambda b, i: (b, 0, jnp.maximum(i * (S // HALO) - 1, 0))),
            pl.BlockSpec((1, CIN, S), lambda b, i: (b, 0, i)),
            pl.BlockSpec((1, CIN, HALO),
                         lambda b, i: (b, 0, jnp.minimum((i + 1) * (S // HALO),
                                                         HW // HALO - 1))),
            pl.BlockSpec((1, 1, S), lambda b, i: (b * NT + i, 0, 0)),
        ],
        out_specs=pl.BlockSpec((1, COUT, S), lambda b, i: (b, 0, i)),
        out_shape=jax.ShapeDtypeStruct((B, COUT, HW), jnp.float32),
        compiler_params=pltpu.CompilerParams(
            dimension_semantics=("parallel", "arbitrary")),
    )(wt, b2, xf, xf, xf, mf)
    return out.reshape(B, COUT, H, W)


# speedup vs baseline: 1.9706x; 1.0265x over previous
import functools

import jax
import jax.numpy as jnp
from jax.experimental import pallas as pl
from jax.experimental.pallas import tpu as pltpu

B, CIN, COUT, H, W, K = 2, 96, 96, 384, 384, 3
HW = H * W
S = 8192
NT = HW // S
HALO = 512


def _conv_body(w_ref, b_ref, xl_ref, xc_ref, xr_ref, m_ref, o_ref):
    i = pl.program_id(1)
    zeros_halo = jnp.zeros((CIN, HALO), jnp.float32)
    xl = jnp.where(i == 0, zeros_halo, xl_ref[0])
    xr = jnp.where(i == NT - 1, zeros_halo, xr_ref[0])
    xcat = jnp.concatenate([xl, xc_ref[0], xr],
                           axis=1).astype(jnp.bfloat16)

    j = jax.lax.broadcasted_iota(jnp.int32, (1, S + 2 * HALO), 1)
    col = (i * S + j + (2 * W - HALO)) % W
    x_m1 = jnp.where(col == W - 1, jnp.bfloat16(0), xcat)
    x_p1 = jnp.where(col == 0, jnp.bfloat16(0), xcat)

    acc = jnp.zeros((COUT, S), jnp.float32)
    for ky in range(K):
        for kx in range(K):
            src = (x_m1, xcat, x_p1)[kx]
            d = (ky - 1) * W + (kx - 1)
            sl = jax.lax.slice(src, (0, HALO + d), (CIN, HALO + d + S))
            acc = acc + jnp.dot(w_ref[ky * K + kx], sl,
                                preferred_element_type=jnp.float32)

    m = m_ref[0].astype(jnp.float32)
    o_ref[0] = (acc + b_ref[...]) * m


@jax.jit
def kernel(x, mask, weight, bias):
    xf = x.reshape(B, CIN, HW)
    mf = mask.reshape(B * NT, 1, S)
    wt = weight.transpose(2, 3, 0, 1).reshape(K * K, COUT, CIN).astype(jnp.bfloat16)
    b2 = bias.reshape(COUT, 1)

    grid = (B, NT)
    out = pl.pallas_call(
        _conv_body,
        grid=grid,
        in_specs=[
            pl.BlockSpec((K * K, COUT, CIN), lambda b, i: (0, 0, 0)),
            pl.BlockSpec((COUT, 1), lambda b, i: (0, 0)),
            pl.BlockSpec((1, CIN, HALO),
                         lambda b, i: (b, 0, jnp.maximum(i * (S // HALO) - 1, 0))),
            pl.BlockSpec((1, CIN, S), lambda b, i: (b, 0, i)),
            pl.BlockSpec((1, CIN, HALO),
                         lambda b, i: (b, 0, jnp.minimum((i + 1) * (S // HALO),
                                                         HW // HALO - 1))),
            pl.BlockSpec((1, 1, S), lambda b, i: (b * NT + i, 0, 0)),
        ],
        out_specs=pl.BlockSpec((1, COUT, S), lambda b, i: (b, 0, i)),
        out_shape=jax.ShapeDtypeStruct((B, COUT, HW), jnp.float32),
        compiler_params=pltpu.CompilerParams(
            dimension_semantics=("parallel", "arbitrary")),
    )(wt, b2, xf, xf, xf, mf)
    return out.reshape(B, COUT, H, W)
